# baseline (speedup 1.0000x reference)
import jax
import jax.numpy as jnp
from jax import lax
from jax.experimental import pallas as pl
from jax.experimental.pallas import tpu as pltpu
from jax.experimental.pallas import tpu_sc as plsc

EMBED = 64
BATCH = 16384
LAM = 1e-05

NC = 2
NS = 16
NW = NC * NS
PW = BATCH // NW
CH = 128
NCH = PW // CH


def _sc_body(h_hbm, r_hbm, p_hbm, n_hbm, ent_hbm, rel_hbm,
             delta_hbm, l2_hbm,
             hidx, ridx, pidx, nidx,
             hbuf, rbuf, pbuf, nbuf,
             dout, l2v, sem):
    wid = lax.axis_index("s") * NC + lax.axis_index("c")
    base = wid * PW

    l2 = jnp.zeros((16,), jnp.float32)
    for c in range(NCH):
        row0 = base + c * CH
        pltpu.sync_copy(h_hbm.at[pl.ds(row0, CH)], hidx.at[c])
        pltpu.sync_copy(r_hbm.at[pl.ds(row0, CH)], ridx.at[c])
        pltpu.sync_copy(p_hbm.at[pl.ds(row0, CH)], pidx.at[c])
        pltpu.sync_copy(n_hbm.at[pl.ds(row0, CH)], nidx.at[c])
        cps = [
            pltpu.async_copy(ent_hbm.at[hidx.at[c]], hbuf, sem),
            pltpu.async_copy(rel_hbm.at[ridx.at[c]], rbuf, sem),
            pltpu.async_copy(ent_hbm.at[pidx.at[c]], pbuf, sem),
            pltpu.async_copy(ent_hbm.at[nidx.at[c]], nbuf, sem),
        ]
        for cp in cps:
            cp.wait()

        def row_body(i, l2c):
            dl = jnp.zeros((16,), jnp.float32)
            for d in range(EMBED // 16):
                sl = pl.ds(16 * d, 16)
                hv = hbuf[i, sl]
                rv = rbuf[i, sl]
                pv = pbuf[i, sl]
                nv = nbuf[i, sl]
                s = hv + rv
                dp = s - pv
                dn = s - nv
                dl = dl + (dp * dp - dn * dn)
                l2c = l2c + hv * hv + rv * rv + pv * pv + nv * nv
            dout[i, :] = dl
            return l2c

        l2 = lax.fori_loop(0, CH, row_body, l2)
        pltpu.sync_copy(dout, delta_hbm.at[pl.ds(row0, CH)])

    l2v[...] = l2
    pltpu.sync_copy(l2v, l2_hbm.at[wid])


_sc_call = pl.kernel(
    _sc_body,
    out_type=[
        jax.ShapeDtypeStruct((BATCH, 16), jnp.float32),
        jax.ShapeDtypeStruct((NW, 16), jnp.float32),
    ],
    mesh=plsc.VectorSubcoreMesh(core_axis_name="c", subcore_axis_name="s"),
    scratch_types=[
        pltpu.VMEM((NCH, CH), jnp.int32),
        pltpu.VMEM((NCH, CH), jnp.int32),
        pltpu.VMEM((NCH, CH), jnp.int32),
        pltpu.VMEM((NCH, CH), jnp.int32),
        pltpu.VMEM((CH, EMBED), jnp.float32),
        pltpu.VMEM((CH, EMBED), jnp.float32),
        pltpu.VMEM((CH, EMBED), jnp.float32),
        pltpu.VMEM((CH, EMBED), jnp.float32),
        pltpu.VMEM((CH, 16), jnp.float32),
        pltpu.VMEM((16,), jnp.float32),
        pltpu.SemaphoreType.DMA,
    ],
    compiler_params=pltpu.CompilerParams(use_tc_tiling_on_sc=False),
)


def _tc_body(x_ref, l2_ref, out_ref):
    x = x_ref[...]
    g = lax.broadcasted_iota(jnp.int32, (128, 8), 0) // 16
    c = lax.broadcasted_iota(jnp.int32, (128, 8), 1)
    m = (g == c).astype(jnp.float32)
    y = lax.dot_general(x, m, (((1,), (0,)), ((), ())),
                        preferred_element_type=jnp.float32)
    sp = jnp.maximum(y, 0.0) + jnp.log1p(jnp.exp(-jnp.abs(y)))
    l2tot = jnp.sum(l2_ref[...])
    loss = jnp.sum(sp) / BATCH + LAM * (l2tot / (2.0 * BATCH))
    out_ref[...] = jnp.full((1, 1), 0.0, jnp.float32) + loss


def kernel(h, r, pos_t, neg_t, entity_embed, relation_embed):
    delta, l2p = _sc_call(h, r, pos_t, neg_t, entity_embed, relation_embed)
    x = delta.reshape(BATCH // 8, 128)
    l2x = l2p.reshape(NW * 16 // 128, 128)
    out = pl.pallas_call(
        _tc_body,
        out_shape=jax.ShapeDtypeStruct((1, 1), jnp.float32),
    )(x, l2x)
    return out[0, 0]


# speedup vs baseline: 1.6073x; 1.6073x over previous
import jax
import jax.numpy as jnp
from jax import lax
from jax.experimental import pallas as pl
from jax.experimental.pallas import tpu as pltpu
from jax.experimental.pallas import tpu_sc as plsc

EMBED = 64
BATCH = 16384
LAM = 1e-05

NC = 2
NS = 16
NW = NC * NS
PW = BATCH // NW
CH = 128
NCH = PW // CH


def _sc_body(h_hbm, r_hbm, p_hbm, n_hbm, ent_hbm, rel_hbm,
             delta_hbm, l2_hbm,
             hidx, ridx, pidx, nidx,
             hbuf, rbuf, pbuf, nbuf,
             dout, l2v, sem):
    wid = lax.axis_index("s") * NC + lax.axis_index("c")
    base = wid * PW

    l2 = jnp.zeros((16,), jnp.float32)
    for c in range(NCH):
        row0 = base + c * CH
        pltpu.sync_copy(h_hbm.at[pl.ds(row0, CH)], hidx.at[c])
        pltpu.sync_copy(r_hbm.at[pl.ds(row0, CH)], ridx.at[c])
        pltpu.sync_copy(p_hbm.at[pl.ds(row0, CH)], pidx.at[c])
        pltpu.sync_copy(n_hbm.at[pl.ds(row0, CH)], nidx.at[c])

        def fire(g, carry):
            hv = hidx[c, pl.ds(16 * g, 16)]
            rv = ridx[c, pl.ds(16 * g, 16)]
            pv = pidx[c, pl.ds(16 * g, 16)]
            nv = nidx[c, pl.ds(16 * g, 16)]
            for l in range(16):
                i = 16 * g + l
                pltpu.make_async_copy(ent_hbm.at[hv[l]], hbuf.at[i], sem).start()
                pltpu.make_async_copy(rel_hbm.at[rv[l]], rbuf.at[i], sem).start()
                pltpu.make_async_copy(ent_hbm.at[pv[l]], pbuf.at[i], sem).start()
                pltpu.make_async_copy(ent_hbm.at[nv[l]], nbuf.at[i], sem).start()
            return carry

        lax.fori_loop(0, CH // 16, fire, 0)
        pltpu.make_async_copy(ent_hbm.at[pl.ds(0, CH)], hbuf, sem).wait()
        pltpu.make_async_copy(rel_hbm.at[pl.ds(0, CH)], rbuf, sem).wait()
        pltpu.make_async_copy(ent_hbm.at[pl.ds(0, CH)], pbuf, sem).wait()
        pltpu.make_async_copy(ent_hbm.at[pl.ds(0, CH)], nbuf, sem).wait()

        def row_body(i, l2c):
            dl = jnp.zeros((16,), jnp.float32)
            for d in range(EMBED // 16):
                sl = pl.ds(16 * d, 16)
                hv = hbuf[i, sl]
                rv = rbuf[i, sl]
                pv = pbuf[i, sl]
                nv = nbuf[i, sl]
                s = hv + rv
                dp = s - pv
                dn = s - nv
                dl = dl + (dp * dp - dn * dn)
                l2c = l2c + hv * hv + rv * rv + pv * pv + nv * nv
            dout[i, :] = dl
            return l2c

        l2 = lax.fori_loop(0, CH, row_body, l2)
        pltpu.sync_copy(dout, delta_hbm.at[pl.ds(row0, CH)])

    l2v[...] = l2
    pltpu.sync_copy(l2v, l2_hbm.at[wid])


_sc_call = pl.kernel(
    _sc_body,
    out_type=[
        jax.ShapeDtypeStruct((BATCH, 16), jnp.float32),
        jax.ShapeDtypeStruct((NW, 16), jnp.float32),
    ],
    mesh=plsc.VectorSubcoreMesh(core_axis_name="c", subcore_axis_name="s"),
    scratch_types=[
        pltpu.VMEM((NCH, CH), jnp.int32),
        pltpu.VMEM((NCH, CH), jnp.int32),
        pltpu.VMEM((NCH, CH), jnp.int32),
        pltpu.VMEM((NCH, CH), jnp.int32),
        pltpu.VMEM((CH, EMBED), jnp.float32),
        pltpu.VMEM((CH, EMBED), jnp.float32),
        pltpu.VMEM((CH, EMBED), jnp.float32),
        pltpu.VMEM((CH, EMBED), jnp.float32),
        pltpu.VMEM((CH, 16), jnp.float32),
        pltpu.VMEM((16,), jnp.float32),
        pltpu.SemaphoreType.DMA,
    ],
)


def _tc_body(x_ref, l2_ref, out_ref):
    x = x_ref[...]
    g = lax.broadcasted_iota(jnp.int32, (128, 8), 0) // 16
    c = lax.broadcasted_iota(jnp.int32, (128, 8), 1)
    m = (g == c).astype(jnp.float32)
    y = lax.dot_general(x, m, (((1,), (0,)), ((), ())),
                        preferred_element_type=jnp.float32)
    sp = jnp.maximum(y, 0.0) + jnp.log1p(jnp.exp(-jnp.abs(y)))
    l2tot = jnp.sum(l2_ref[...])
    loss = jnp.sum(sp) / BATCH + LAM * (l2tot / (2.0 * BATCH))
    out_ref[...] = jnp.full((1, 1), 0.0, jnp.float32) + loss


def kernel(h, r, pos_t, neg_t, entity_embed, relation_embed):
    delta, l2p = _sc_call(h, r, pos_t, neg_t, entity_embed, relation_embed)
    x = delta.reshape(BATCH // 8, 128)
    l2x = l2p.reshape(NW * 16 // 128, 128)
    out = pl.pallas_call(
        _tc_body,
        out_shape=jax.ShapeDtypeStruct((1, 1), jnp.float32),
    )(x, l2x)
    return out[0, 0]
